# baseline (speedup 1.0000x reference)
import functools

import jax
import jax.numpy as jnp
from jax import lax
from jax.experimental import pallas as pl
from jax.experimental.pallas import tpu as pltpu
from jax.experimental.pallas import tpu_sc as plsc

NUM_Q = 4
K = 1024
D = 256
BETA = 0.25
BN = 512
NCH = 2


def _stage_kernel(first, r_ref, xqp_ref, cb_ref, idx_ref, rout_ref, loss_ref,
                  cbm2t_scr, cbn_scr):
    i = pl.program_id(0)

    @pl.when(i == 0)
    def _prep():
        cb = cb_ref[...]
        cbm2t_scr[...] = -2.0 * cb.T
        cbn_scr[...] = jnp.sum(cb * cb, axis=1, keepdims=True).T

    H = BN // NCH
    loss_acc = jnp.zeros((), jnp.float32)
    for c in range(NCH):
        sl = slice(c * H, (c + 1) * H)
        r = r_ref[sl, :]
        if not first:
            xq = xqp_ref[sl, :]
            t = xq - r
            x_res = r + t
            loss_acc = loss_acc + jnp.sum(t * t)
            r = r - x_res
            rout_ref[sl, :] = r
        rn = jnp.sum(r * r, axis=1, keepdims=True)
        sm2 = jnp.dot(r, cbm2t_scr[...], preferred_element_type=jnp.float32)
        d = (rn + cbn_scr[...]) + sm2
        idx_ref[sl, :] = lax.argmin(d, axis=1, index_dtype=jnp.int32)[:, None]

    if not first:
        @pl.when(i == 0)
        def _():
            loss_ref[...] = jnp.zeros((1, 1), jnp.float32)

        loss_ref[...] += loss_acc[None, None]


def _final_kernel(x_ref, r_ref, xqp_ref, xq_ref, loss_ref):
    i = pl.program_id(0)
    r = r_ref[...]
    t = xqp_ref[...] - r
    x_res = r + t
    r4 = r - x_res
    xq_ref[...] = x_ref[...] - r4

    @pl.when(i == 0)
    def _():
        loss_ref[...] = jnp.zeros((1, 1), jnp.float32)

    loss_ref[...] += jnp.sum(t * t)[None, None]


def _tc_stage(first, r, xq_prev, cb):
    n = r.shape[0]
    nb = n // BN
    specs = [pl.BlockSpec((BN, D), lambda i: (i, 0)),
             pl.BlockSpec((BN, D), lambda i: (i, 0)),
             pl.BlockSpec((K, D), lambda i: (0, 0))]
    out_specs = [pl.BlockSpec((BN, 1), lambda i: (i, 0)),
                 pl.BlockSpec((BN, D), lambda i: (i, 0)),
                 pl.BlockSpec((1, 1), lambda i: (0, 0))]
    out_shape = [jax.ShapeDtypeStruct((n, 1), jnp.int32),
                 jax.ShapeDtypeStruct((n, D), jnp.float32),
                 jax.ShapeDtypeStruct((1, 1), jnp.float32)]
    res = pl.pallas_call(
        functools.partial(_stage_kernel, first),
        grid=(nb,),
        in_specs=specs,
        out_specs=out_specs,
        out_shape=out_shape,
        scratch_shapes=[pltpu.VMEM((D, K), jnp.float32),
                        pltpu.VMEM((1, K), jnp.float32)],
    )(r, xq_prev, cb)
    idx, rout, loss = res
    return idx.reshape(n), rout, loss[0, 0]


def _sc_gather(cb, idx):
    n = idx.shape[0]
    info = plsc.get_sparse_core_info()
    nw = info.num_cores * info.num_subcores
    b_per_w = n // nw
    mesh = plsc.VectorSubcoreMesh(core_axis_name="c", subcore_axis_name="s")

    @functools.partial(
        pl.kernel, mesh=mesh,
        out_type=jax.ShapeDtypeStruct((n, D), jnp.float32),
        scratch_types=[
            pltpu.VMEM((b_per_w,), jnp.int32),
            pltpu.VMEM((b_per_w, D), jnp.float32),
            pltpu.SemaphoreType.DMA,
        ],
    )
    def k(table_hbm, idx_hbm, out_hbm, idx_v, rows_v, sem):
        wid = lax.axis_index("s") * info.num_cores + lax.axis_index("c")
        base = wid * b_per_w
        pltpu.sync_copy(idx_hbm.at[pl.ds(base, b_per_w)], idx_v)
        pltpu.async_copy(table_hbm.at[idx_v], rows_v, sem).wait()
        pltpu.sync_copy(rows_v, out_hbm.at[pl.ds(base, b_per_w)])

    return k(cb, idx)


def _final(x, r3, xq3):
    n = x.shape[0]
    nb = n // BN
    xq, loss = pl.pallas_call(
        _final_kernel,
        grid=(nb,),
        in_specs=[pl.BlockSpec((BN, D), lambda i: (i, 0))] * 3,
        out_specs=[pl.BlockSpec((BN, D), lambda i: (i, 0)),
                   pl.BlockSpec((1, 1), lambda i: (0, 0))],
        out_shape=[jax.ShapeDtypeStruct((n, D), jnp.float32),
                   jax.ShapeDtypeStruct((1, 1), jnp.float32)],
    )(x, r3, xq3)
    return xq, loss[0, 0]


def kernel(x, codebooks):
    n = x.shape[0]
    r = x
    idxs = []
    losses = []
    xq_prev = x
    for s in range(NUM_Q):
        idx, rout, lossp = _tc_stage(s == 0, r, xq_prev, codebooks[s])
        if s > 0:
            losses.append(lossp)
            r = rout
        idxs.append(idx)
        xq_prev = _sc_gather(codebooks[s], idx)
    xq, loss3 = _final(x, r, xq_prev)
    losses.append(loss3)
    scale = (1.0 + BETA) / (NUM_Q * n * D)
    loss = (((losses[0] + losses[1]) + losses[2]) + losses[3]) * scale
    idx_out = jnp.stack(idxs, axis=1)
    return xq, loss, idx_out


# speedup vs baseline: 2.0018x; 2.0018x over previous
import functools

import jax
import jax.numpy as jnp
from jax.experimental import pallas as pl
from jax.experimental.pallas import tpu as pltpu

NUM_Q = 4
K = 1024
D = 256
BETA = 0.25
BN = 512


def _rvq_kernel(x_ref, cb_ref, xq_ref, loss_ref, idx_ref,
                cbm2t_scr, cbn_scr, csplit_scr, *, n_total):
    i = pl.program_id(0)

    @pl.when(i == 0)
    def _prep():
        for s in range(NUM_Q):
            cb = cb_ref[s]
            cbm2t_scr[s] = -2.0 * cb.T
            cbn_scr[s] = jnp.sum(cb * cb, axis=1, keepdims=True).T
            c1 = cb.astype(jnp.bfloat16)
            rem = cb - c1.astype(jnp.float32)
            c2 = rem.astype(jnp.bfloat16)
            c3 = (rem - c2.astype(jnp.float32)).astype(jnp.bfloat16)
            csplit_scr[s, :, :D] = c1
            csplit_scr[s, :, D:2 * D] = c2
            csplit_scr[s, :, 2 * D:] = c3

    NCH = 2
    H = BN // NCH
    rs = [x_ref[c * H:(c + 1) * H, :] for c in range(NCH)]
    rns = [jnp.sum(r * r, axis=1, keepdims=True) for r in rs]
    xq_accs = [jnp.zeros_like(r) for r in rs]
    loss_acc = jnp.zeros((), jnp.float32)
    idx_cols = [[] for _ in range(NCH)]
    for s in range(NUM_Q):
        for h in range(NCH):
            r = rs[h]
            sm2 = jnp.dot(r, cbm2t_scr[s],
                          preferred_element_type=jnp.float32)
            d = (rns[h] + cbn_scr[s]) + sm2
            m = jnp.min(d, axis=1, keepdims=True)
            iota = jax.lax.broadcasted_iota(jnp.int32, d.shape, 1)
            idx = jnp.min(jnp.where(d == m, iota, K), axis=1, keepdims=True)
            onehot = (iota == idx).astype(jnp.bfloat16)
            xq3 = jnp.dot(onehot, csplit_scr[s],
                          preferred_element_type=jnp.float32)
            xq = (xq3[:, :D] + xq3[:, D:2 * D]) + xq3[:, 2 * D:]
            t = xq - r
            x_res = r + t
            loss_acc = loss_acc + jnp.sum(t * t)
            rs[h] = r - x_res
            rns[h] = jnp.sum(rs[h] * rs[h], axis=1, keepdims=True)
            xq_accs[h] = xq_accs[h] + x_res
            idx_cols[h].append(idx)

    for c in range(NCH):
        xq_ref[c * H:(c + 1) * H, :] = xq_accs[c]
        idx_ref[c * H:(c + 1) * H, :] = jnp.concatenate(idx_cols[c], axis=1)

    scale = (1.0 + BETA) / (NUM_Q * n_total * D)

    @pl.when(i == 0)
    def _():
        loss_ref[...] = jnp.zeros((1, 1), jnp.float32)

    loss_ref[...] += (loss_acc * scale)[None, None]


def kernel(x, codebooks):
    n = x.shape[0]
    nb = n // BN

    xq, loss, idx = pl.pallas_call(
        functools.partial(_rvq_kernel, n_total=n),
        grid=(nb,),
        in_specs=[
            pl.BlockSpec((BN, D), lambda i: (i, 0)),
            pl.BlockSpec((NUM_Q, K, D), lambda i: (0, 0, 0)),
        ],
        out_specs=[
            pl.BlockSpec((BN, D), lambda i: (i, 0)),
            pl.BlockSpec((1, 1), lambda i: (0, 0)),
            pl.BlockSpec((BN, NUM_Q), lambda i: (i, 0)),
        ],
        out_shape=[
            jax.ShapeDtypeStruct((n, D), jnp.float32),
            jax.ShapeDtypeStruct((1, 1), jnp.float32),
            jax.ShapeDtypeStruct((n, NUM_Q), jnp.int32),
        ],
        scratch_shapes=[
            pltpu.VMEM((NUM_Q, D, K), jnp.float32),
            pltpu.VMEM((NUM_Q, 1, K), jnp.float32),
            pltpu.VMEM((NUM_Q, K, 3 * D), jnp.bfloat16),
        ],
    )(x, codebooks)
    return xq, loss[0, 0], idx
